# baseline (speedup 1.0000x reference)
import functools

import jax
import jax.numpy as jnp
from jax import lax
from jax.experimental import pallas as pl
from jax.experimental.pallas import tpu as pltpu
from jax.experimental.pallas import tpu_sc as plsc

HID = 128
_NC, _NS = 2, 16
_NW = _NC * _NS
_LN_EPS = 1e-5


def _contract(x, w):
    return lax.dot_general(x, w, (((1,), (1,)), ((), ())),
                           preferred_element_type=jnp.float32)


def _make_encode_body(blk, iblk, nreal, nseg, tmask):
    def body(h_ref, a_ref, w1_ref, b1_ref, w2_ref, g1_ref, bb1_ref,
             w3_ref, g2_ref, bb2_ref, e0_ref, e1_ref, le_ref, pe_ref,
             xval_ref, m_ref, oidx_ref, iidx_ref):
        i = pl.program_id(0)
        h = h_ref[...]
        a_col = a_ref[...].reshape(blk, 1)
        hext = jnp.concatenate([h, a_col], axis=1)
        t = jnp.maximum(_contract(hext, w1_ref[...]) + b1_ref[...], 0.0)
        u = _contract(t, w2_ref[...])
        mu = jnp.mean(u, axis=1, keepdims=True)
        var = jnp.mean(u * u, axis=1, keepdims=True) - mu * mu
        xv = (u - mu) * lax.rsqrt(var + _LN_EPS) * g1_ref[...] + bb1_ref[...]
        xval_ref[...] = xv
        y = _contract(xv, w3_ref[...])
        mu2 = jnp.mean(y, axis=1, keepdims=True)
        var2 = jnp.mean(y * y, axis=1, keepdims=True) - mu2 * mu2
        m = (y - mu2) * lax.rsqrt(var2 + _LN_EPS) * g2_ref[...] + bb2_ref[...]
        m_ref[...] = m.reshape(4 * blk, HID)

        absrow = (lax.broadcasted_iota(jnp.int32, (iblk, _CH), 0) + i * iblk)
        lane = lax.broadcasted_iota(jnp.int32, (iblk, _CH), 1)
        real = absrow < nreal
        e0 = e0_ref[...].reshape(iblk, _CH)
        e1 = e1_ref[...].reshape(iblk, _CH)
        og = e1 * 4 + le_ref[...] * 2 + pe_ref[...]
        opad = (absrow * 2731 + lane * 997) & tmask
        oidx_ref[...] = jnp.where(real, og, opad)
        iidx_ref[...] = jnp.where(real, e0, nseg + lane)
    return body


def _encode(h_val, assign, W1, b1, W2, ln1_g, ln1_b, W3, ln2_g, ln2_b,
            e0f, e1f, LEr, PEr, ntot):
    n = h_val.shape[0]
    nreal = LEr.shape[0]
    blk = 1000
    assert n % blk == 0
    grid = (n // blk,)
    iblk = ntot // (n // blk)
    assert iblk * (n // blk) == ntot and iblk % 8 == 0
    tmask = 16383
    assert tmask < 4 * n
    row = lambda i: (i, 0)
    full = lambda i: (0, 0)
    idx_spec = pl.BlockSpec((iblk, _CH), row)
    x_val, m4, oidx, iidx = pl.pallas_call(
        _make_encode_body(blk, iblk, nreal, n, tmask),
        grid=grid,
        in_specs=[
            pl.BlockSpec((blk, HID), row),
            pl.BlockSpec((1, 1, blk), lambda i: (i, 0, 0)),
            pl.BlockSpec((HID, HID + 1), full),
            pl.BlockSpec((1, HID), full),
            pl.BlockSpec((HID, HID), full),
            pl.BlockSpec((1, HID), full),
            pl.BlockSpec((1, HID), full),
            pl.BlockSpec((4 * HID, HID), full),
            pl.BlockSpec((1, 4 * HID), full),
            pl.BlockSpec((1, 4 * HID), full),
            pl.BlockSpec((iblk * _CH,), lambda i: (i,)),
            pl.BlockSpec((iblk * _CH,), lambda i: (i,)),
            idx_spec,
            idx_spec,
        ],
        out_specs=[
            pl.BlockSpec((blk, HID), row),
            pl.BlockSpec((4 * blk, HID), row),
            idx_spec,
            idx_spec,
        ],
        out_shape=[
            jax.ShapeDtypeStruct((n, HID), jnp.float32),
            jax.ShapeDtypeStruct((4 * n, HID), jnp.float32),
            jax.ShapeDtypeStruct((ntot, _CH), jnp.int32),
            jax.ShapeDtypeStruct((ntot, _CH), jnp.int32),
        ],
    )(h_val, assign.reshape(n // blk, 1, blk), W1, b1.reshape(1, HID),
      W2, ln1_g.reshape(1, HID), ln1_b.reshape(1, HID),
      W3, ln2_g.reshape(1, 4 * HID), ln2_b.reshape(1, 4 * HID),
      e0f, e1f, LEr, PEr)
    return x_val, m4, oidx, iidx


_CH = 128


def _sc_pad_geometry(E, nseg):
    nreal = -(-E // _CH)
    cpw = (-(-nreal // _NW) + 7) & ~7
    return cpw, cpw * _NW


@functools.lru_cache(maxsize=None)
def _make_sc(E, nseg):
    assert E % _CH == 0
    cpw, ntot = _sc_pad_geometry(E, nseg)
    assert cpw % 2 == 0 and cpw >= 4
    seg_pw = (nseg // _NS) & ~7
    seg_rem = nseg - seg_pw * _NS
    assert seg_rem % 8 == 0
    nz_full = seg_pw // _CH
    nz_rem = seg_pw - nz_full * _CH

    mesh = plsc.VectorSubcoreMesh(core_axis_name="c", subcore_axis_name="s",
                                  num_cores=_NC, num_subcores=_NS)

    nstage = 2
    assert cpw % nstage == 0
    hpw = cpw // nstage
    assert hpw % 8 == 0 and hpw % 2 == 0 and hpw >= 4

    scratch = [
        pltpu.VMEM((hpw, _CH), jnp.int32),
        pltpu.VMEM((hpw, _CH), jnp.int32),
        pltpu.VMEM((_CH, HID), jnp.float32),
        pltpu.VMEM((_CH, HID), jnp.float32),
        pltpu.VMEM_SHARED((nseg + 128, HID), jnp.float32),
        pltpu.SemaphoreType.DMA,
        pltpu.SemaphoreType.DMA,
    ]

    @functools.partial(
        pl.kernel,
        mesh=mesh,
        out_type=jax.ShapeDtypeStruct((_NC * nseg, HID), jnp.float32),
        scratch_types=scratch,
    )
    def sc_kernel(mval_hbm, oidx_hbm, iidx_hbm, out_hbm,
                  vo, vi, rows0, rows1, acc, sem0, sem1):
        cid = lax.axis_index("c")
        sid = lax.axis_index("s")
        wid = sid * _NC + cid

        def zrow(r, _):
            for k in range(HID // 16):
                rows0[r, pl.ds(k * 16, 16)] = jnp.zeros((16,), jnp.float32)
            return 0
        lax.fori_loop(0, _CH, zrow, 0)
        base_seg = pl.multiple_of(sid * seg_pw, 8)
        zcopies = []
        for t in range(nz_full):
            zcopies.append(pltpu.make_async_copy(
                rows0, acc.at[pl.ds(base_seg + t * _CH, _CH)], sem0))
        if nz_rem:
            zcopies.append(pltpu.make_async_copy(
                rows0.at[pl.ds(0, nz_rem)],
                acc.at[pl.ds(base_seg + nz_full * _CH, nz_rem)], sem0))
        for zc in zcopies:
            zc.start()
        if seg_rem:
            @pl.when(sid == 0)
            def _():
                pltpu.sync_copy(rows0.at[pl.ds(0, seg_rem)],
                                acc.at[pl.ds(_NS * seg_pw, seg_rem)])

        def gstart(j, rows_b, sem_b):
            pltpu.async_copy(mval_hbm.at[vo.at[j]], rows_b, sem_b)

        def gwait(j, rows_b, sem_b):
            pltpu.make_async_copy(mval_hbm.at[vo.at[j]], rows_b, sem_b).wait()

        def scat(j, rows_b):
            pltpu.sync_copy(rows_b, acc.at[vi.at[j]], add=True)

        row0 = pl.multiple_of(wid * cpw, 8)
        for h in range(nstage):
            base_row = pl.multiple_of(row0 + h * hpw, 8)
            pltpu.sync_copy(oidx_hbm.at[pl.ds(base_row, hpw)], vo)
            pltpu.sync_copy(iidx_hbm.at[pl.ds(base_row, hpw)], vi)
            if h == 0:
                for zc in zcopies:
                    zc.wait()
                plsc.subcore_barrier()
            gstart(0, rows0, sem0)

            def body(p, _):
                j1 = 2 * p + 1
                gstart(j1, rows1, sem1)
                gwait(2 * p, rows0, sem0)
                scat(2 * p, rows0)
                gstart(j1 + 1, rows0, sem0)
                gwait(j1, rows1, sem1)
                scat(j1, rows1)
                return 0
            lax.fori_loop(0, hpw // 2 - 1, body, 0)

            gstart(hpw - 1, rows1, sem1)
            gwait(hpw - 2, rows0, sem0)
            scat(hpw - 2, rows0)
            gwait(hpw - 1, rows1, sem1)
            scat(hpw - 1, rows1)

        plsc.subcore_barrier()
        out_base = pl.multiple_of(cid * nseg + base_seg, 8)
        pltpu.sync_copy(acc.at[pl.ds(base_seg, seg_pw)],
                        out_hbm.at[pl.ds(out_base, seg_pw)])
        if seg_rem:
            @pl.when(sid == 0)
            def _():
                rem_base = pl.multiple_of(cid * nseg + _NS * seg_pw, 8)
                pltpu.sync_copy(acc.at[pl.ds(_NS * seg_pw, seg_rem)],
                                out_hbm.at[pl.ds(rem_base, seg_rem)])

    return sc_kernel


def _add_body(p_ref, o_ref):
    o_ref[...] = p_ref[0] + p_ref[1]


def _add_partials(partials, n):
    blk = 2000
    return pl.pallas_call(
        _add_body,
        grid=(n // blk,),
        in_specs=[pl.BlockSpec((2, blk, HID), lambda i: (0, i, 0))],
        out_specs=pl.BlockSpec((blk, HID), lambda i: (i, 0)),
        out_shape=jax.ShapeDtypeStruct((n, HID), jnp.float32),
    )(partials)


def kernel(h_val, assign, cst_edges, LE, PE, num_val, num_cst,
           W1, b1, W2, ln1_g, ln1_b, W3, ln2_g, ln2_b):
    n = h_val.shape[0]
    E = cst_edges.shape[1]
    cpw, ntot = _sc_pad_geometry(E, n)
    nreal = E // _CH
    e0f = cst_edges[0].astype(jnp.int32)
    e1f = cst_edges[1].astype(jnp.int32)
    LEr = LE.astype(jnp.int32).reshape(nreal, _CH)
    PEr = PE.astype(jnp.int32).reshape(nreal, _CH)
    x_val, m_val, oidx, iidx = _encode(h_val, assign, W1, b1, W2,
                                       ln1_g, ln1_b, W3, ln2_g, ln2_b,
                                       e0f, e1f, LEr, PEr, ntot)
    partials = _make_sc(E, n)(m_val, oidx, iidx)
    r_cst = _add_partials(partials.reshape(2, n, HID), n)
    return (r_cst, x_val)


# speedup vs baseline: 1.0611x; 1.0611x over previous
import functools

import jax
import jax.numpy as jnp
from jax import lax
from jax.experimental import pallas as pl
from jax.experimental.pallas import tpu as pltpu
from jax.experimental.pallas import tpu_sc as plsc

HID = 128
_NC, _NS = 2, 16
_NW = _NC * _NS
_LN_EPS = 1e-5


def _contract(x, w):
    return lax.dot_general(x, w, (((1,), (1,)), ((), ())),
                           preferred_element_type=jnp.float32)


def _make_encode_body(blk, iblk, nreal, nseg, tmask):
    def body(h_ref, a_ref, w1_ref, b1_ref, w2_ref, g1_ref, bb1_ref,
             w3_ref, g2_ref, bb2_ref, e01_ref, le_ref, pe_ref,
             xval_ref, m_ref, oidx_ref, iidx_ref):
        i = pl.program_id(0)
        h = h_ref[...]
        a_col = a_ref[...].reshape(blk, 1)
        hext = jnp.concatenate([h, a_col], axis=1)
        t = jnp.maximum(_contract(hext, w1_ref[...]) + b1_ref[...], 0.0)
        u = _contract(t, w2_ref[...])
        mu = jnp.mean(u, axis=1, keepdims=True)
        var = jnp.mean(u * u, axis=1, keepdims=True) - mu * mu
        xv = (u - mu) * lax.rsqrt(var + _LN_EPS) * g1_ref[...] + bb1_ref[...]
        xval_ref[...] = xv
        y = _contract(xv, w3_ref[...])
        mu2 = jnp.mean(y, axis=1, keepdims=True)
        var2 = jnp.mean(y * y, axis=1, keepdims=True) - mu2 * mu2
        m = (y - mu2) * lax.rsqrt(var2 + _LN_EPS) * g2_ref[...] + bb2_ref[...]
        m_ref[...] = m.reshape(4 * blk, HID)

        absrow = (lax.broadcasted_iota(jnp.int32, (iblk, _CH), 0) + i * iblk)
        lane = lax.broadcasted_iota(jnp.int32, (iblk, _CH), 1)
        real = absrow < nreal
        e01 = e01_ref[...]
        og = e01[1] * 4 + le_ref[...] * 2 + pe_ref[...]
        opad = (absrow * 2731 + lane * 997) & tmask
        oidx_ref[...] = jnp.where(real, og, opad)
        iidx_ref[...] = jnp.where(real, e01[0], nseg + lane)
    return body


def _encode(h_val, assign, W1, b1, W2, ln1_g, ln1_b, W3, ln2_g, ln2_b,
            c3, LEr, PEr, ntot):
    n = h_val.shape[0]
    nreal = c3.shape[1]
    blk = 1000
    assert n % blk == 0
    grid = (n // blk,)
    iblk = ntot // (n // blk)
    assert iblk * (n // blk) == ntot and iblk % 8 == 0
    tmask = 16383
    assert tmask < 4 * n
    row = lambda i: (i, 0)
    full = lambda i: (0, 0)
    idx_spec = pl.BlockSpec((iblk, _CH), row)
    x_val, m4, oidx, iidx = pl.pallas_call(
        _make_encode_body(blk, iblk, nreal, n, tmask),
        grid=grid,
        in_specs=[
            pl.BlockSpec((blk, HID), row),
            pl.BlockSpec((1, 1, blk), lambda i: (i, 0, 0)),
            pl.BlockSpec((HID, HID + 1), full),
            pl.BlockSpec((1, HID), full),
            pl.BlockSpec((HID, HID), full),
            pl.BlockSpec((1, HID), full),
            pl.BlockSpec((1, HID), full),
            pl.BlockSpec((4 * HID, HID), full),
            pl.BlockSpec((1, 4 * HID), full),
            pl.BlockSpec((1, 4 * HID), full),
            pl.BlockSpec((2, iblk, _CH), lambda i: (0, i, 0)),
            idx_spec,
            idx_spec,
        ],
        out_specs=[
            pl.BlockSpec((blk, HID), row),
            pl.BlockSpec((4 * blk, HID), row),
            idx_spec,
            idx_spec,
        ],
        out_shape=[
            jax.ShapeDtypeStruct((n, HID), jnp.float32),
            jax.ShapeDtypeStruct((4 * n, HID), jnp.float32),
            jax.ShapeDtypeStruct((ntot, _CH), jnp.int32),
            jax.ShapeDtypeStruct((ntot, _CH), jnp.int32),
        ],
    )(h_val, assign.reshape(n // blk, 1, blk), W1, b1.reshape(1, HID),
      W2, ln1_g.reshape(1, HID), ln1_b.reshape(1, HID),
      W3, ln2_g.reshape(1, 4 * HID), ln2_b.reshape(1, 4 * HID),
      c3, LEr, PEr)
    return x_val, m4, oidx, iidx


_CH = 128


def _sc_pad_geometry(E, nseg):
    nreal = -(-E // _CH)
    cpw = (-(-nreal // _NW) + 7) & ~7
    return cpw, cpw * _NW


@functools.lru_cache(maxsize=None)
def _make_sc(E, nseg):
    assert E % _CH == 0
    cpw, ntot = _sc_pad_geometry(E, nseg)
    assert cpw % 2 == 0 and cpw >= 4
    seg_pw = (nseg // _NS) & ~7
    seg_rem = nseg - seg_pw * _NS
    assert seg_rem % 8 == 0
    nz_full = seg_pw // _CH
    nz_rem = seg_pw - nz_full * _CH

    mesh = plsc.VectorSubcoreMesh(core_axis_name="c", subcore_axis_name="s",
                                  num_cores=_NC, num_subcores=_NS)

    nstage = 2
    assert cpw % nstage == 0
    hpw = cpw // nstage
    assert hpw % 8 == 0 and hpw % 2 == 0 and hpw >= 4

    scratch = [
        pltpu.VMEM((hpw, _CH), jnp.int32),
        pltpu.VMEM((hpw, _CH), jnp.int32),
        pltpu.VMEM((_CH, HID), jnp.float32),
        pltpu.VMEM((_CH, HID), jnp.float32),
        pltpu.VMEM_SHARED((nseg + 128, HID), jnp.float32),
        pltpu.SemaphoreType.DMA,
        pltpu.SemaphoreType.DMA,
    ]

    @functools.partial(
        pl.kernel,
        mesh=mesh,
        out_type=jax.ShapeDtypeStruct((_NC * nseg, HID), jnp.float32),
        scratch_types=scratch,
    )
    def sc_kernel(mval_hbm, oidx_hbm, iidx_hbm, out_hbm,
                  vo, vi, rows0, rows1, acc, sem0, sem1):
        cid = lax.axis_index("c")
        sid = lax.axis_index("s")
        wid = sid * _NC + cid

        def zrow(r, _):
            for k in range(HID // 16):
                rows0[r, pl.ds(k * 16, 16)] = jnp.zeros((16,), jnp.float32)
            return 0
        lax.fori_loop(0, _CH, zrow, 0)
        base_seg = pl.multiple_of(sid * seg_pw, 8)
        zcopies = []
        for t in range(nz_full):
            zcopies.append(pltpu.make_async_copy(
                rows0, acc.at[pl.ds(base_seg + t * _CH, _CH)], sem0))
        if nz_rem:
            zcopies.append(pltpu.make_async_copy(
                rows0.at[pl.ds(0, nz_rem)],
                acc.at[pl.ds(base_seg + nz_full * _CH, nz_rem)], sem0))
        for zc in zcopies:
            zc.start()
        if seg_rem:
            @pl.when(sid == 0)
            def _():
                pltpu.sync_copy(rows0.at[pl.ds(0, seg_rem)],
                                acc.at[pl.ds(_NS * seg_pw, seg_rem)])

        def gstart(j, rows_b, sem_b):
            pltpu.async_copy(mval_hbm.at[vo.at[j]], rows_b, sem_b)

        def gwait(j, rows_b, sem_b):
            pltpu.make_async_copy(mval_hbm.at[vo.at[j]], rows_b, sem_b).wait()

        def scat(j, rows_b):
            pltpu.sync_copy(rows_b, acc.at[vi.at[j]], add=True)

        row0 = pl.multiple_of(wid * cpw, 8)
        for h in range(nstage):
            base_row = pl.multiple_of(row0 + h * hpw, 8)
            pltpu.sync_copy(oidx_hbm.at[pl.ds(base_row, hpw)], vo)
            pltpu.sync_copy(iidx_hbm.at[pl.ds(base_row, hpw)], vi)
            if h == 0:
                for zc in zcopies:
                    zc.wait()
                plsc.subcore_barrier()
            gstart(0, rows0, sem0)

            def body(p, _):
                j1 = 2 * p + 1
                gstart(j1, rows1, sem1)
                gwait(2 * p, rows0, sem0)
                scat(2 * p, rows0)
                gstart(j1 + 1, rows0, sem0)
                gwait(j1, rows1, sem1)
                scat(j1, rows1)
                return 0
            lax.fori_loop(0, hpw // 2 - 1, body, 0)

            gstart(hpw - 1, rows1, sem1)
            gwait(hpw - 2, rows0, sem0)
            scat(hpw - 2, rows0)
            gwait(hpw - 1, rows1, sem1)
            scat(hpw - 1, rows1)

        plsc.subcore_barrier()
        out_base = pl.multiple_of(cid * nseg + base_seg, 8)
        pltpu.sync_copy(acc.at[pl.ds(base_seg, seg_pw)],
                        out_hbm.at[pl.ds(out_base, seg_pw)])
        if seg_rem:
            @pl.when(sid == 0)
            def _():
                rem_base = pl.multiple_of(cid * nseg + _NS * seg_pw, 8)
                pltpu.sync_copy(acc.at[pl.ds(_NS * seg_pw, seg_rem)],
                                out_hbm.at[pl.ds(rem_base, seg_rem)])

    return sc_kernel


def _add_body(p_ref, o_ref):
    o_ref[...] = p_ref[0] + p_ref[1]


def _add_partials(partials, n):
    blk = 2000
    return pl.pallas_call(
        _add_body,
        grid=(n // blk,),
        in_specs=[pl.BlockSpec((2, blk, HID), lambda i: (0, i, 0))],
        out_specs=pl.BlockSpec((blk, HID), lambda i: (i, 0)),
        out_shape=jax.ShapeDtypeStruct((n, HID), jnp.float32),
    )(partials)


def kernel(h_val, assign, cst_edges, LE, PE, num_val, num_cst,
           W1, b1, W2, ln1_g, ln1_b, W3, ln2_g, ln2_b):
    n = h_val.shape[0]
    E = cst_edges.shape[1]
    cpw, ntot = _sc_pad_geometry(E, n)
    nreal = E // _CH
    c3 = cst_edges.astype(jnp.int32).reshape(2, nreal, _CH)
    LEr = LE.astype(jnp.int32).reshape(nreal, _CH)
    PEr = PE.astype(jnp.int32).reshape(nreal, _CH)
    x_val, m_val, oidx, iidx = _encode(h_val, assign, W1, b1, W2,
                                       ln1_g, ln1_b, W3, ln2_g, ln2_b,
                                       c3, LEr, PEr, ntot)
    partials = _make_sc(E, n)(m_val, oidx, iidx)
    r_cst = _add_partials(partials.reshape(2, n, HID), n)
    return (r_cst, x_val)
